# Initial kernel scaffold; baseline (speedup 1.0000x reference)
import jax
import jax.numpy as jnp
from jax.experimental import pallas as pl


def kernel(frame_feat, mark_embeddings, W_frame, b_frame, gamma, frame_masks):
    raise NotImplementedError("write your pallas kernel here")


# speedup vs baseline: 7.0602x; 7.0602x over previous
import functools

import jax
import jax.numpy as jnp
from jax.experimental import pallas as pl
from jax.experimental.pallas import tpu as pltpu

T, D, K, H, W = 128, 768, 16, 512, 512


def _hist_body(mask_ref, counts_ref):
    t = pl.program_id(0)
    m = mask_ref[...]
    sums = []
    for k in range(1, K + 1):
        sums.append(jnp.sum((m == k).astype(jnp.float32)))
    counts_ref[t] = jnp.stack(sums)


def _dense_body(ff_ref, marks_ref, w_ref, b_ref, gamma_ref, counts_ref, out_ref):
    p = jax.lax.dot_general(
        marks_ref[...], w_ref[...], (((1,), (1,)), ((), ())),
        preferred_element_type=jnp.float32)
    p = p + b_ref[...]
    counts = counts_ref[...]
    sm = jax.lax.dot_general(
        counts, p, (((1,), (0,)), ((), ())),
        preferred_element_type=jnp.float32)
    wsum = jnp.sum(counts, axis=1, keepdims=True) + 1e-6
    out_ref[...] = ff_ref[...] + gamma_ref[0] * sm / wsum


@jax.jit
def kernel(frame_feat, mark_embeddings, W_frame, b_frame, gamma, frame_masks):
    counts = pl.pallas_call(
        _hist_body,
        grid=(T,),
        in_specs=[pl.BlockSpec((1, H, W), lambda t: (t, 0, 0))],
        out_specs=pl.BlockSpec((T, K), lambda t: (0, 0)),
        out_shape=jax.ShapeDtypeStruct((T, K), jnp.float32),
    )(frame_masks)

    out = pl.pallas_call(
        _dense_body,
        in_specs=[
            pl.BlockSpec((T, D), lambda: (0, 0)),
            pl.BlockSpec((K, D), lambda: (0, 0)),
            pl.BlockSpec((D, D), lambda: (0, 0)),
            pl.BlockSpec((1, D), lambda: (0, 0)),
            pl.BlockSpec(memory_space=pltpu.SMEM),
            pl.BlockSpec((T, K), lambda: (0, 0)),
        ],
        out_specs=pl.BlockSpec((T, D), lambda: (0, 0)),
        out_shape=jax.ShapeDtypeStruct((T, D), jnp.float32),
    )(frame_feat, mark_embeddings, W_frame, b_frame.reshape(1, D),
      jnp.reshape(gamma, (1,)), counts)
    return out
